# baseline (speedup 1.0000x reference)
import numpy as np
import jax
import jax.numpy as jnp
from jax.experimental import pallas as pl
from jax.experimental.pallas import tpu as pltpu

_B = 16384
_K = 100
_SPAN = 999999

_ROT = ((13, 15, 26, 6), (17, 29, 16, 24))


_K2_0, _K2_1 = 64467757, 2916123636


def _neg_sample_block(o_ref, *, rows_per_block):
    base = pl.program_id(0) * (rows_per_block * _K)
    shape = (rows_per_block, _K)
    i = (jax.lax.broadcasted_iota(jnp.int32, shape, 0) * _K
         + jax.lax.broadcasted_iota(jnp.int32, shape, 1)
         + base).astype(jnp.uint32)

    ks0 = jnp.uint32(_K2_0)
    ks1 = jnp.uint32(_K2_1)
    ks2 = jnp.uint32(_K2_0 ^ _K2_1 ^ 0x1BD11BDA)
    ks = (ks0, ks1, ks2)

    x0 = jnp.full(shape, ks0, jnp.uint32)
    x1 = i + ks1
    for r in range(5):
        for d in _ROT[r % 2]:
            x0 = x0 + x1
            x1 = (x1 << d) | (x1 >> (32 - d))
            x1 = x0 ^ x1
        x0 = x0 + ks[(r + 1) % 3]
        x1 = x1 + ks[(r + 2) % 3] + jnp.uint32(r + 1)
    bits = x0 ^ x1

    hi = (bits >> 16).astype(jnp.int32).astype(jnp.float32)
    lo = (bits & jnp.uint32(0xFFFF)).astype(jnp.int32).astype(jnp.float32)
    xf = hi * 65536.0 + lo
    q = (xf * (1.0 / float(_SPAN))).astype(jnp.int32).astype(jnp.uint32)
    r = bits - q * jnp.uint32(_SPAN)
    r = jnp.where(r >= jnp.uint32(0x80000000), r + jnp.uint32(_SPAN), r)
    r = jnp.where(r >= jnp.uint32(_SPAN), r - jnp.uint32(_SPAN), r)
    o_ref[...] = (r + jnp.uint32(1)).astype(jnp.int32)


def kernel(k, pos_targets):
    del k, pos_targets
    rows_per_block = 2048
    grid = (_B // rows_per_block,)
    from functools import partial
    return pl.pallas_call(
        partial(_neg_sample_block, rows_per_block=rows_per_block),
        grid=grid,
        out_shape=jax.ShapeDtypeStruct((_B, _K), jnp.int32),
        out_specs=pl.BlockSpec((rows_per_block, _K), lambda b: (b, 0)),
    )()


# speedup vs baseline: 1.0557x; 1.0557x over previous
import numpy as np
import jax
import jax.numpy as jnp
from jax.experimental import pallas as pl
from jax.experimental.pallas import tpu as pltpu

_B = 16384
_K = 100
_SPAN = 999999

_ROT = ((13, 15, 26, 6), (17, 29, 16, 24))


_K2_0, _K2_1 = 64467757, 2916123636


def _key_schedule():
    m = (1 << 32) - 1
    ks = (_K2_0, _K2_1, _K2_0 ^ _K2_1 ^ 0x1BD11BDA)
    return tuple(
        (ks[(r + 1) % 3], (ks[(r + 2) % 3] + r + 1) & m) for r in range(5)
    )


_KS = _key_schedule()


def _neg_sample_block(o_ref, *, rows_per_block):
    base = pl.program_id(0) * (rows_per_block * _K)
    shape = (rows_per_block, _K)
    i = (jax.lax.broadcasted_iota(jnp.int32, shape, 0) * _K
         + jax.lax.broadcasted_iota(jnp.int32, shape, 1)
         + base).astype(jnp.uint32)

    x0 = jnp.full(shape, _K2_0, jnp.uint32)
    x1 = i + jnp.uint32(_K2_1)
    for r in range(5):
        for d in _ROT[r % 2]:
            x0 = x0 + x1
            x1 = (x1 << d) | (x1 >> (32 - d))
            x1 = x0 ^ x1
        x0 = x0 + jnp.uint32(_KS[r][0])
        x1 = x1 + jnp.uint32(_KS[r][1])
    bits = x0 ^ x1

    qf = (bits >> 1).astype(jnp.int32).astype(jnp.float32) * (2.0 / float(_SPAN))
    q = qf.astype(jnp.int32).astype(jnp.uint32)
    r = bits - q * jnp.uint32(_SPAN)
    r = jnp.where(r >= jnp.uint32(0x80000000), r + jnp.uint32(_SPAN), r)
    r = jnp.where(r >= jnp.uint32(_SPAN), r - jnp.uint32(_SPAN), r)
    o_ref[...] = (r + jnp.uint32(1)).astype(jnp.int32)


def kernel(k, pos_targets):
    del k, pos_targets
    rows_per_block = 2048
    grid = (_B // rows_per_block,)
    from functools import partial
    return pl.pallas_call(
        partial(_neg_sample_block, rows_per_block=rows_per_block),
        grid=grid,
        out_shape=jax.ShapeDtypeStruct((_B, _K), jnp.int32),
        out_specs=pl.BlockSpec((rows_per_block, _K), lambda b: (b, 0)),
    )()
